# baseline (speedup 1.0000x reference)
import functools

import numpy as np
import jax
import jax.numpy as jnp
from jax.experimental import pallas as pl
from jax.experimental.pallas import tpu as pltpu

OUT_H = 512
OUT_W = 512
SRC_HW = 128
MH = 28
MW = 28
NPAD = 128
BAND = 128
STUFF_OFFSET = 90.0
MASK_THR = 0.5
SCORE_THR = 0.05
VOID_ENC = 6553600.0


def _interp_matrix(out_size: int, in_size: int) -> np.ndarray:
    i = np.arange(out_size, dtype=np.float32)
    src = (i + 0.5) * (in_size / out_size) - 0.5
    j = np.arange(in_size, dtype=np.float32)
    w = np.maximum(0.0, 1.0 - np.abs(j[None, :] - src[:, None])).astype(np.float32)
    w = w / np.sum(w, axis=1, keepdims=True)
    return w.astype(np.float32)


def _panoptic_kernel(nreal, nchan, sa_ref, sb_ref, cls_ref, m_ref, seg_ref,
                     colperm_ref, scores_sm, boxes_sm, cat_ref, inst_ref,
                     encv_ref):
    f32 = jnp.float32
    bf16 = jnp.bfloat16
    b = pl.program_id(0)

    si = sa_ref[0]
    sj = sb_ref[0]
    ii = jax.lax.broadcasted_iota(jnp.int32, (NPAD, NPAD), 0)
    jj = jax.lax.broadcasted_iota(jnp.int32, (NPAD, NPAD), 1)
    beats = (sj > si) | ((sj == si) & (jj < ii))
    ranks = jnp.sum(beats.astype(f32), axis=1, keepdims=True)
    clsv = cls_ref[0]
    dv = jax.lax.broadcasted_iota(jnp.int32, (NPAD, 1), 0).astype(f32)
    validv = si > SCORE_THR
    encv_ref[...] = jnp.where(validv, ranks * 65536.0 + clsv * 128.0 + dv,
                              VOID_ENC)

    srcm1 = SRC_HW - 1

    def ch_body(c, carry):
        bests, bidxs = carry
        x = seg_ref[0, c]
        xm1 = jnp.concatenate([x[:, 0:1], x[:, 0:srcm1]], axis=1)
        xp1 = jnp.concatenate([x[:, 1:SRC_HW], x[:, srcm1:SRC_HW]], axis=1)
        t = jnp.concatenate(
            [0.375 * xm1 + 0.625 * x,
             0.125 * xm1 + 0.875 * x,
             0.875 * x + 0.125 * xp1,
             0.625 * x + 0.375 * xp1], axis=1)
        tm1 = jnp.concatenate([t[0:1], t[0:srcm1]], axis=0)
        tp1 = jnp.concatenate([t[1:SRC_HW], t[srcm1:SRC_HW]], axis=0)
        phases = (0.375 * tm1 + 0.625 * t,
                  0.125 * tm1 + 0.875 * t,
                  0.875 * t + 0.125 * tp1,
                  0.625 * t + 0.375 * tp1)
        cf = c.astype(f32)
        upds = tuple(v > bb for v, bb in zip(phases, bests))
        bests = tuple(jnp.where(u, v, bb)
                      for u, v, bb in zip(upds, phases, bests))
        bidxs = tuple(jnp.where(u, cf, bi) for u, bi in zip(upds, bidxs))
        return bests, bidxs

    best0 = tuple(jnp.full((SRC_HW, OUT_W), -jnp.inf, f32) for _ in range(4))
    bidx0 = tuple(jnp.zeros((SRC_HW, OUT_W), f32) for _ in range(4))
    _, bidxs = jax.lax.fori_loop(0, nchan, ch_body, (best0, bidx0))
    segf_blk = jnp.stack(bidxs, axis=1).reshape(OUT_H, OUT_W)
    segf = jnp.dot(segf_blk, colperm_ref[...], preferred_element_type=f32)

    yi = jax.lax.broadcasted_iota(jnp.int32, (BAND, MH), 0).astype(f32)
    ky = jax.lax.broadcasted_iota(jnp.int32, (BAND, MH), 1).astype(f32)
    jx = jax.lax.broadcasted_iota(jnp.int32, (MW, OUT_W), 0).astype(f32)
    xi = jax.lax.broadcasted_iota(jnp.int32, (MW, OUT_W), 1).astype(f32)

    bands = []
    for band in range(OUT_H // BAND):
        y0 = band * BAND

        def det_body(d, encmin_b, y0=y0):
            ymin_i = boxes_sm[b, d, 0]
            ymaxc_i = jnp.minimum(boxes_sm[b, d, 2] + 1, OUT_H)
            hit = ((scores_sm[b, d] > SCORE_THR)
                   & (ymaxc_i > y0) & (ymin_i < y0 + BAND))

            def paint(em):
                ymin = ymin_i.astype(f32)
                ymaxc = ymaxc_i.astype(f32)
                xmin = boxes_sm[b, d, 1].astype(f32)
                xmaxc = jnp.minimum(boxes_sm[b, d, 3] + 1, OUT_W).astype(f32)
                bh = jnp.maximum(ymaxc - ymin, 1.0)
                bw = jnp.maximum(xmaxc - xmin, 1.0)
                ya = yi + float(y0)
                fy = ((ya - ymin) + 0.5) * (MH / bh)
                sy = jnp.clip(jnp.floor(fy), 0.0, float(MH - 1))
                oy = ((ky == sy) & (ya >= ymin) & (ya < ymaxc)).astype(bf16)
                fx = ((xi - xmin) + 0.5) * (MW / bw)
                sx = jnp.clip(jnp.floor(fx), 0.0, float(MW - 1))
                oxt = ((jx == sx) & (xi >= xmin) & (xi < xmaxc)).astype(bf16)
                bm = (m_ref[0, d] > MASK_THR).astype(bf16)
                q = jnp.dot(oy, bm, preferred_element_type=f32)
                cov = jnp.dot(q.astype(bf16), oxt,
                              preferred_element_type=f32)
                e = encv_ref[pl.ds(d, 1), :]
                return jnp.minimum(em, jnp.where(cov > 0.5, e, VOID_ENC))

            return jax.lax.cond(hit, paint, lambda em: em, encmin_b)

        bands.append(jax.lax.fori_loop(
            0, nreal, det_body, jnp.full((BAND, OUT_W), VOID_ENC, f32)))

    encmin = jnp.concatenate(bands, axis=0)

    found = encmin < VOID_ENC
    r = jnp.floor(encmin * (1.0 / 65536.0))
    rem = encmin - r * 65536.0
    cls = jnp.floor(rem * (1.0 / 128.0))
    dd = rem - cls * 128.0
    catf = jnp.where(found, cls, 0.0)
    instf = jnp.where(found, dd + 1.0, -1.0)
    stuff = (segf != 0.0) & (segf != 1.0)
    catf = jnp.where((~found) & stuff, segf + STUFF_OFFSET, catf)
    cat_ref[0] = catf.astype(jnp.int32)
    inst_ref[0] = instf.astype(jnp.int32)


def _run(detection_scores, detection_classes, detection_boxes,
         detection_masks, segmentation_outputs, interpret):
    B, N = detection_scores.shape
    C = segmentation_outputs.shape[-1]

    pad = NPAD - N
    scores = jnp.pad(detection_scores, ((0, 0), (0, pad)),
                     constant_values=-1.0)
    classes = jnp.pad(detection_classes, ((0, 0), (0, pad)))
    boxes = jnp.pad(detection_boxes, ((0, 0), (0, pad), (0, 0)))
    masks = jnp.pad(detection_masks, ((0, 0), (0, pad), (0, 0), (0, 0)))
    boxes_i = boxes.astype(jnp.int32)

    sa = scores.reshape(B, NPAD, 1)
    sb = scores.reshape(B, 1, NPAD)
    cls_a = classes.reshape(B, NPAD, 1)
    seg_t = jnp.transpose(segmentation_outputs, (0, 3, 1, 2))

    cp = np.zeros((OUT_W, OUT_W), np.float32)
    qq, jj = np.meshgrid(np.arange(4), np.arange(SRC_HW), indexing='ij')
    cp[SRC_HW * qq.ravel() + jj.ravel(), 4 * jj.ravel() + qq.ravel()] = 1.0
    colperm = jnp.asarray(cp)

    grid = (B,)
    kern = functools.partial(_panoptic_kernel, N, C)
    cat, inst = pl.pallas_call(
        kern,
        grid=grid,
        in_specs=[
            pl.BlockSpec((1, NPAD, 1), lambda b: (b, 0, 0)),
            pl.BlockSpec((1, 1, NPAD), lambda b: (b, 0, 0)),
            pl.BlockSpec((1, NPAD, 1), lambda b: (b, 0, 0)),
            pl.BlockSpec((1, NPAD, MH, MW), lambda b: (b, 0, 0, 0)),
            pl.BlockSpec((1, C, SRC_HW, SRC_HW), lambda b: (b, 0, 0, 0)),
            pl.BlockSpec((OUT_W, OUT_W), lambda b: (0, 0)),
            pl.BlockSpec(memory_space=pltpu.SMEM),
            pl.BlockSpec(memory_space=pltpu.SMEM),
        ],
        out_specs=[
            pl.BlockSpec((1, OUT_H, OUT_W), lambda b: (b, 0, 0)),
            pl.BlockSpec((1, OUT_H, OUT_W), lambda b: (b, 0, 0)),
        ],
        out_shape=[
            jax.ShapeDtypeStruct((B, OUT_H, OUT_W), jnp.int32),
            jax.ShapeDtypeStruct((B, OUT_H, OUT_W), jnp.int32),
        ],
        scratch_shapes=[pltpu.VMEM((NPAD, 1), jnp.float32)],
        interpret=interpret,
    )(sa, sb, cls_a, masks, seg_t, colperm, scores, boxes_i)
    return cat, inst


def kernel(detection_scores, detection_classes, detection_boxes,
           detection_masks, segmentation_outputs):
    return _run(detection_scores, detection_classes, detection_boxes,
                detection_masks, segmentation_outputs, False)


# speedup vs baseline: 1.1517x; 1.1517x over previous
import functools

import numpy as np
import jax
import jax.numpy as jnp
from jax.experimental import pallas as pl
from jax.experimental.pallas import tpu as pltpu

OUT_H = 512
OUT_W = 512
SRC_HW = 128
MH = 28
MW = 28
NPAD = 128
BAND = 128
STUFF_OFFSET = 90.0
MASK_THR = 0.5
SCORE_THR = 0.05
VOID_ENC = 6553600.0


def _interp_matrix(out_size: int, in_size: int) -> np.ndarray:
    i = np.arange(out_size, dtype=np.float32)
    src = (i + 0.5) * (in_size / out_size) - 0.5
    j = np.arange(in_size, dtype=np.float32)
    w = np.maximum(0.0, 1.0 - np.abs(j[None, :] - src[:, None])).astype(np.float32)
    w = w / np.sum(w, axis=1, keepdims=True)
    return w.astype(np.float32)


def _panoptic_kernel(nreal, nchan, sa_ref, sb_ref, cls_ref, m_ref, seg_ref,
                     colperm_ref, scores_sm, boxes_sm, cat_ref, inst_ref,
                     encv_ref, encmin_ref):
    f32 = jnp.float32
    bf16 = jnp.bfloat16
    b = pl.program_id(0)

    si = sa_ref[0]
    sj = sb_ref[0]
    ii = jax.lax.broadcasted_iota(jnp.int32, (NPAD, NPAD), 0)
    jj = jax.lax.broadcasted_iota(jnp.int32, (NPAD, NPAD), 1)
    beats = (sj > si) | ((sj == si) & (jj < ii))
    ranks = jnp.sum(beats.astype(f32), axis=1, keepdims=True)
    clsv = cls_ref[0]
    dv = jax.lax.broadcasted_iota(jnp.int32, (NPAD, 1), 0).astype(f32)
    validv = si > SCORE_THR
    encv_ref[...] = jnp.where(validv, ranks * 65536.0 + clsv * 128.0 + dv,
                              VOID_ENC)

    srcm1 = SRC_HW - 1

    def ch_body(c, carry):
        bests, bidxs = carry
        x = seg_ref[0, c]
        xm1 = jnp.concatenate([x[:, 0:1], x[:, 0:srcm1]], axis=1)
        xp1 = jnp.concatenate([x[:, 1:SRC_HW], x[:, srcm1:SRC_HW]], axis=1)
        t = jnp.concatenate(
            [0.375 * xm1 + 0.625 * x,
             0.125 * xm1 + 0.875 * x,
             0.875 * x + 0.125 * xp1,
             0.625 * x + 0.375 * xp1], axis=1)
        tm1 = jnp.concatenate([t[0:1], t[0:srcm1]], axis=0)
        tp1 = jnp.concatenate([t[1:SRC_HW], t[srcm1:SRC_HW]], axis=0)
        phases = (0.375 * tm1 + 0.625 * t,
                  0.125 * tm1 + 0.875 * t,
                  0.875 * t + 0.125 * tp1,
                  0.625 * t + 0.375 * tp1)
        cf = c.astype(f32)
        upds = tuple(v > bb for v, bb in zip(phases, bests))
        bests = tuple(jnp.where(u, v, bb)
                      for u, v, bb in zip(upds, phases, bests))
        bidxs = tuple(jnp.where(u, cf, bi) for u, bi in zip(upds, bidxs))
        return bests, bidxs

    best0 = tuple(jnp.full((SRC_HW, OUT_W), -jnp.inf, f32) for _ in range(4))
    bidx0 = tuple(jnp.zeros((SRC_HW, OUT_W), f32) for _ in range(4))
    _, bidxs = jax.lax.fori_loop(0, nchan, ch_body, (best0, bidx0))
    segf_blk = jnp.stack(bidxs, axis=1).reshape(OUT_H, OUT_W)
    segf = jnp.dot(segf_blk, colperm_ref[...], preferred_element_type=f32)

    yi = jax.lax.broadcasted_iota(jnp.int32, (BAND, MH), 0).astype(f32)
    ky = jax.lax.broadcasted_iota(jnp.int32, (BAND, MH), 1).astype(f32)
    jx = jax.lax.broadcasted_iota(jnp.int32, (MW, OUT_W), 0).astype(f32)
    xi = jax.lax.broadcasted_iota(jnp.int32, (MW, OUT_W), 1).astype(f32)

    encmin_ref[...] = jnp.full((OUT_H, OUT_W), VOID_ENC, f32)

    for band in range(OUT_H // BAND):
        y0 = band * BAND

        def det_body(d, _, y0=y0):
            ymin_i = boxes_sm[b, d, 0]
            ymaxc_i = jnp.minimum(boxes_sm[b, d, 2] + 1, OUT_H)
            hit = ((scores_sm[b, d] > SCORE_THR)
                   & (ymaxc_i > y0) & (ymin_i < y0 + BAND))

            @pl.when(hit)
            def _():
                ymin = ymin_i.astype(f32)
                ymaxc = ymaxc_i.astype(f32)
                xmin = boxes_sm[b, d, 1].astype(f32)
                xmaxc = jnp.minimum(boxes_sm[b, d, 3] + 1, OUT_W).astype(f32)
                bh = jnp.maximum(ymaxc - ymin, 1.0)
                bw = jnp.maximum(xmaxc - xmin, 1.0)
                ya = yi + float(y0)
                fy = ((ya - ymin) + 0.5) * (MH / bh)
                sy = jnp.clip(jnp.floor(fy), 0.0, float(MH - 1))
                oy = ((ky == sy) & (ya >= ymin) & (ya < ymaxc)).astype(bf16)
                fx = ((xi - xmin) + 0.5) * (MW / bw)
                sx = jnp.clip(jnp.floor(fx), 0.0, float(MW - 1))
                oxt = ((jx == sx) & (xi >= xmin) & (xi < xmaxc)).astype(bf16)
                bm = (m_ref[0, d] > MASK_THR).astype(bf16)
                q = jnp.dot(oy, bm, preferred_element_type=f32)
                cov = jnp.dot(q.astype(bf16), oxt,
                              preferred_element_type=f32)
                e = encv_ref[pl.ds(d, 1), :]
                cur = encmin_ref[y0:y0 + BAND, :]
                encmin_ref[y0:y0 + BAND, :] = jnp.minimum(
                    cur, jnp.where(cov > 0.5, e, VOID_ENC))
            return 0

        jax.lax.fori_loop(0, nreal, det_body, 0)

    encmin = encmin_ref[...]

    found = encmin < VOID_ENC
    r = jnp.floor(encmin * (1.0 / 65536.0))
    rem = encmin - r * 65536.0
    cls = jnp.floor(rem * (1.0 / 128.0))
    dd = rem - cls * 128.0
    catf = jnp.where(found, cls, 0.0)
    instf = jnp.where(found, dd + 1.0, -1.0)
    stuff = (segf != 0.0) & (segf != 1.0)
    catf = jnp.where((~found) & stuff, segf + STUFF_OFFSET, catf)
    cat_ref[0] = catf.astype(jnp.int32)
    inst_ref[0] = instf.astype(jnp.int32)


def _run(detection_scores, detection_classes, detection_boxes,
         detection_masks, segmentation_outputs, interpret):
    B, N = detection_scores.shape
    C = segmentation_outputs.shape[-1]

    pad = NPAD - N
    scores = jnp.pad(detection_scores, ((0, 0), (0, pad)),
                     constant_values=-1.0)
    classes = jnp.pad(detection_classes, ((0, 0), (0, pad)))
    boxes = jnp.pad(detection_boxes, ((0, 0), (0, pad), (0, 0)))
    masks = jnp.pad(detection_masks, ((0, 0), (0, pad), (0, 0), (0, 0)))
    boxes_i = boxes.astype(jnp.int32)

    sa = scores.reshape(B, NPAD, 1)
    sb = scores.reshape(B, 1, NPAD)
    cls_a = classes.reshape(B, NPAD, 1)
    seg_t = jnp.transpose(segmentation_outputs, (0, 3, 1, 2))

    cp = np.zeros((OUT_W, OUT_W), np.float32)
    qq, jj = np.meshgrid(np.arange(4), np.arange(SRC_HW), indexing='ij')
    cp[SRC_HW * qq.ravel() + jj.ravel(), 4 * jj.ravel() + qq.ravel()] = 1.0
    colperm = jnp.asarray(cp)

    grid = (B,)
    kern = functools.partial(_panoptic_kernel, N, C)
    cat, inst = pl.pallas_call(
        kern,
        grid=grid,
        in_specs=[
            pl.BlockSpec((1, NPAD, 1), lambda b: (b, 0, 0)),
            pl.BlockSpec((1, 1, NPAD), lambda b: (b, 0, 0)),
            pl.BlockSpec((1, NPAD, 1), lambda b: (b, 0, 0)),
            pl.BlockSpec((1, NPAD, MH, MW), lambda b: (b, 0, 0, 0)),
            pl.BlockSpec((1, C, SRC_HW, SRC_HW), lambda b: (b, 0, 0, 0)),
            pl.BlockSpec((OUT_W, OUT_W), lambda b: (0, 0)),
            pl.BlockSpec(memory_space=pltpu.SMEM),
            pl.BlockSpec(memory_space=pltpu.SMEM),
        ],
        out_specs=[
            pl.BlockSpec((1, OUT_H, OUT_W), lambda b: (b, 0, 0)),
            pl.BlockSpec((1, OUT_H, OUT_W), lambda b: (b, 0, 0)),
        ],
        out_shape=[
            jax.ShapeDtypeStruct((B, OUT_H, OUT_W), jnp.int32),
            jax.ShapeDtypeStruct((B, OUT_H, OUT_W), jnp.int32),
        ],
        scratch_shapes=[pltpu.VMEM((NPAD, 1), jnp.float32),
                        pltpu.VMEM((OUT_H, OUT_W), jnp.float32)],
        interpret=interpret,
    )(sa, sb, cls_a, masks, seg_t, colperm, scores, boxes_i)
    return cat, inst


def kernel(detection_scores, detection_classes, detection_boxes,
           detection_masks, segmentation_outputs):
    return _run(detection_scores, detection_classes, detection_boxes,
                detection_masks, segmentation_outputs, False)


# speedup vs baseline: 1.5188x; 1.3187x over previous
import functools

import numpy as np
import jax
import jax.numpy as jnp
from jax.experimental import pallas as pl
from jax.experimental.pallas import tpu as pltpu

OUT_H = 512
OUT_W = 512
SRC_HW = 128
MH = 28
MW = 28
NPAD = 128
BAND = 128
STUFF_OFFSET = 90.0
MASK_THR = 0.5
SCORE_THR = 0.05
VOID_ENC = 6553600.0


def _interp_matrix(out_size: int, in_size: int) -> np.ndarray:
    i = np.arange(out_size, dtype=np.float32)
    src = (i + 0.5) * (in_size / out_size) - 0.5
    j = np.arange(in_size, dtype=np.float32)
    w = np.maximum(0.0, 1.0 - np.abs(j[None, :] - src[:, None])).astype(np.float32)
    w = w / np.sum(w, axis=1, keepdims=True)
    return w.astype(np.float32)


def _panoptic_kernel(nreal, nchan, sa_ref, sb_ref, cls_ref, m_ref, seg_ref,
                     colperm_ref, scores_sm, boxes_sm, cat_ref, inst_ref,
                     encv_ref, encmin_ref):
    f32 = jnp.float32
    bf16 = jnp.bfloat16
    b = pl.program_id(0)

    si = sa_ref[0]
    sj = sb_ref[0]
    ii = jax.lax.broadcasted_iota(jnp.int32, (NPAD, NPAD), 0)
    jj = jax.lax.broadcasted_iota(jnp.int32, (NPAD, NPAD), 1)
    beats = (sj > si) | ((sj == si) & (jj < ii))
    ranks = jnp.sum(beats.astype(f32), axis=1, keepdims=True)
    clsv = cls_ref[0]
    dv = jax.lax.broadcasted_iota(jnp.int32, (NPAD, 1), 0).astype(f32)
    validv = si > SCORE_THR
    encv_ref[...] = jnp.where(validv, ranks * 65536.0 + clsv * 128.0 + dv,
                              VOID_ENC)

    srcm1 = SRC_HW - 1

    def ch_body(c, carry):
        bests, bidxs = carry
        x = seg_ref[0, c]
        xm1 = jnp.concatenate([x[:, 0:1], x[:, 0:srcm1]], axis=1)
        xp1 = jnp.concatenate([x[:, 1:SRC_HW], x[:, srcm1:SRC_HW]], axis=1)
        t = jnp.concatenate(
            [0.375 * xm1 + 0.625 * x,
             0.125 * xm1 + 0.875 * x,
             0.875 * x + 0.125 * xp1,
             0.625 * x + 0.375 * xp1], axis=1)
        tm1 = jnp.concatenate([t[0:1], t[0:srcm1]], axis=0)
        tp1 = jnp.concatenate([t[1:SRC_HW], t[srcm1:SRC_HW]], axis=0)
        phases = (0.375 * tm1 + 0.625 * t,
                  0.125 * tm1 + 0.875 * t,
                  0.875 * t + 0.125 * tp1,
                  0.625 * t + 0.375 * tp1)
        cf = c.astype(f32)
        upds = tuple(v > bb for v, bb in zip(phases, bests))
        bests = tuple(jnp.where(u, v, bb)
                      for u, v, bb in zip(upds, phases, bests))
        bidxs = tuple(jnp.where(u, cf, bi) for u, bi in zip(upds, bidxs))
        return bests, bidxs

    best0 = tuple(jnp.full((SRC_HW, OUT_W), -jnp.inf, f32) for _ in range(4))
    bidx0 = tuple(jnp.zeros((SRC_HW, OUT_W), f32) for _ in range(4))
    _, bidxs = jax.lax.fori_loop(0, nchan, ch_body, (best0, bidx0))
    segf_blk = jnp.stack(bidxs, axis=1).reshape(OUT_H, OUT_W)
    segf = jnp.dot(segf_blk, colperm_ref[...], preferred_element_type=f32)

    yi = jax.lax.broadcasted_iota(jnp.int32, (BAND, MH), 0).astype(f32)
    ky = jax.lax.broadcasted_iota(jnp.int32, (BAND, MH), 1).astype(f32)
    jx = jax.lax.broadcasted_iota(jnp.int32, (MW, OUT_W), 0).astype(f32)
    xi = jax.lax.broadcasted_iota(jnp.int32, (MW, OUT_W), 1).astype(f32)

    yi = jax.lax.broadcasted_iota(jnp.int32, (OUT_H, MH), 0).astype(f32)
    ky = jax.lax.broadcasted_iota(jnp.int32, (OUT_H, MH), 1).astype(f32)
    jx = jax.lax.broadcasted_iota(jnp.int32, (MW, OUT_W), 0).astype(f32)
    xi = jax.lax.broadcasted_iota(jnp.int32, (MW, OUT_W), 1).astype(f32)

    def cand(d):
        ymin = boxes_sm[b, d, 0].astype(f32)
        ymaxc = jnp.minimum(boxes_sm[b, d, 2] + 1, OUT_H).astype(f32)
        xmin = boxes_sm[b, d, 1].astype(f32)
        xmaxc = jnp.minimum(boxes_sm[b, d, 3] + 1, OUT_W).astype(f32)
        bh = jnp.maximum(ymaxc - ymin, 1.0)
        bw = jnp.maximum(xmaxc - xmin, 1.0)
        fy = ((yi - ymin) + 0.5) * (MH / bh)
        sy = jnp.clip(jnp.floor(fy), 0.0, float(MH - 1))
        oy = ((ky == sy) & (yi >= ymin) & (yi < ymaxc)).astype(bf16)
        fx = ((xi - xmin) + 0.5) * (MW / bw)
        sx = jnp.clip(jnp.floor(fx), 0.0, float(MW - 1))
        oxt = ((jx == sx) & (xi >= xmin) & (xi < xmaxc)).astype(bf16)
        bm = (m_ref[0, d] > MASK_THR).astype(bf16)
        q = jnp.dot(oy, bm, preferred_element_type=f32)
        cov = jnp.dot(q.astype(bf16), oxt,
                      preferred_element_type=f32)
        e = encv_ref[pl.ds(d, 1), :]
        return jnp.where(cov > 0.5, e, VOID_ENC)

    half = (nreal + 1) // 2

    def det_body(d, encmin):
        return jnp.minimum(encmin, jnp.minimum(cand(d), cand(d + half)))

    encmin = jax.lax.fori_loop(
        0, half, det_body, jnp.full((OUT_H, OUT_W), VOID_ENC, f32))

    found = encmin < VOID_ENC
    r = jnp.floor(encmin * (1.0 / 65536.0))
    rem = encmin - r * 65536.0
    cls = jnp.floor(rem * (1.0 / 128.0))
    dd = rem - cls * 128.0
    catf = jnp.where(found, cls, 0.0)
    instf = jnp.where(found, dd + 1.0, -1.0)
    stuff = (segf != 0.0) & (segf != 1.0)
    catf = jnp.where((~found) & stuff, segf + STUFF_OFFSET, catf)
    cat_ref[0] = catf.astype(jnp.int32)
    inst_ref[0] = instf.astype(jnp.int32)


def _run(detection_scores, detection_classes, detection_boxes,
         detection_masks, segmentation_outputs, interpret):
    B, N = detection_scores.shape
    C = segmentation_outputs.shape[-1]

    pad = NPAD - N
    scores = jnp.pad(detection_scores, ((0, 0), (0, pad)),
                     constant_values=-1.0)
    classes = jnp.pad(detection_classes, ((0, 0), (0, pad)))
    boxes = jnp.pad(detection_boxes, ((0, 0), (0, pad), (0, 0)))
    masks = jnp.pad(detection_masks, ((0, 0), (0, pad), (0, 0), (0, 0)))
    boxes_i = boxes.astype(jnp.int32)

    sa = scores.reshape(B, NPAD, 1)
    sb = scores.reshape(B, 1, NPAD)
    cls_a = classes.reshape(B, NPAD, 1)
    seg_t = jnp.transpose(segmentation_outputs, (0, 3, 1, 2))

    cp = np.zeros((OUT_W, OUT_W), np.float32)
    qq, jj = np.meshgrid(np.arange(4), np.arange(SRC_HW), indexing='ij')
    cp[SRC_HW * qq.ravel() + jj.ravel(), 4 * jj.ravel() + qq.ravel()] = 1.0
    colperm = jnp.asarray(cp)

    grid = (B,)
    kern = functools.partial(_panoptic_kernel, N, C)
    cat, inst = pl.pallas_call(
        kern,
        grid=grid,
        in_specs=[
            pl.BlockSpec((1, NPAD, 1), lambda b: (b, 0, 0)),
            pl.BlockSpec((1, 1, NPAD), lambda b: (b, 0, 0)),
            pl.BlockSpec((1, NPAD, 1), lambda b: (b, 0, 0)),
            pl.BlockSpec((1, NPAD, MH, MW), lambda b: (b, 0, 0, 0)),
            pl.BlockSpec((1, C, SRC_HW, SRC_HW), lambda b: (b, 0, 0, 0)),
            pl.BlockSpec((OUT_W, OUT_W), lambda b: (0, 0)),
            pl.BlockSpec(memory_space=pltpu.SMEM),
            pl.BlockSpec(memory_space=pltpu.SMEM),
        ],
        out_specs=[
            pl.BlockSpec((1, OUT_H, OUT_W), lambda b: (b, 0, 0)),
            pl.BlockSpec((1, OUT_H, OUT_W), lambda b: (b, 0, 0)),
        ],
        out_shape=[
            jax.ShapeDtypeStruct((B, OUT_H, OUT_W), jnp.int32),
            jax.ShapeDtypeStruct((B, OUT_H, OUT_W), jnp.int32),
        ],
        scratch_shapes=[pltpu.VMEM((NPAD, 1), jnp.float32),
                        pltpu.VMEM((OUT_H, OUT_W), jnp.float32)],
        interpret=interpret,
    )(sa, sb, cls_a, masks, seg_t, colperm, scores, boxes_i)
    return cat, inst


def kernel(detection_scores, detection_classes, detection_boxes,
           detection_masks, segmentation_outputs):
    return _run(detection_scores, detection_classes, detection_boxes,
                detection_masks, segmentation_outputs, False)


# speedup vs baseline: 1.6130x; 1.0621x over previous
import functools

import numpy as np
import jax
import jax.numpy as jnp
from jax.experimental import pallas as pl
from jax.experimental.pallas import tpu as pltpu

OUT_H = 512
OUT_W = 512
SRC_HW = 128
MH = 28
MW = 28
NPAD = 128
BAND = 128
STUFF_OFFSET = 90.0
MASK_THR = 0.5
SCORE_THR = 0.05
VOID_ENC = 6553600.0


def _interp_matrix(out_size: int, in_size: int) -> np.ndarray:
    i = np.arange(out_size, dtype=np.float32)
    src = (i + 0.5) * (in_size / out_size) - 0.5
    j = np.arange(in_size, dtype=np.float32)
    w = np.maximum(0.0, 1.0 - np.abs(j[None, :] - src[:, None])).astype(np.float32)
    w = w / np.sum(w, axis=1, keepdims=True)
    return w.astype(np.float32)


def _panoptic_kernel(nreal, nchan, sa_ref, sb_ref, cls_ref, m_ref, seg_ref,
                     colperm_ref, scores_sm, boxes_sm, cat_ref, inst_ref,
                     encv_ref, encmin_ref):
    f32 = jnp.float32
    bf16 = jnp.bfloat16
    b = pl.program_id(0)

    si = sa_ref[0]
    sj = sb_ref[0]
    ii = jax.lax.broadcasted_iota(jnp.int32, (NPAD, NPAD), 0)
    jj = jax.lax.broadcasted_iota(jnp.int32, (NPAD, NPAD), 1)
    beats = (sj > si) | ((sj == si) & (jj < ii))
    ranks = jnp.sum(beats.astype(f32), axis=1, keepdims=True)
    clsv = cls_ref[0]
    dv = jax.lax.broadcasted_iota(jnp.int32, (NPAD, 1), 0).astype(f32)
    validv = si > SCORE_THR
    encv_ref[...] = jnp.where(validv, ranks * 65536.0 + clsv * 128.0 + dv,
                              VOID_ENC)

    srcm1 = SRC_HW - 1

    def ch_body(c, carry):
        bests, bidxs = carry
        x = seg_ref[0, c]
        xm1 = jnp.concatenate([x[:, 0:1], x[:, 0:srcm1]], axis=1)
        xp1 = jnp.concatenate([x[:, 1:SRC_HW], x[:, srcm1:SRC_HW]], axis=1)
        t = jnp.concatenate(
            [0.375 * xm1 + 0.625 * x,
             0.125 * xm1 + 0.875 * x,
             0.875 * x + 0.125 * xp1,
             0.625 * x + 0.375 * xp1], axis=1)
        tm1 = jnp.concatenate([t[0:1], t[0:srcm1]], axis=0)
        tp1 = jnp.concatenate([t[1:SRC_HW], t[srcm1:SRC_HW]], axis=0)
        phases = (0.375 * tm1 + 0.625 * t,
                  0.125 * tm1 + 0.875 * t,
                  0.875 * t + 0.125 * tp1,
                  0.625 * t + 0.375 * tp1)
        cf = c.astype(f32)
        upds = tuple(v > bb for v, bb in zip(phases, bests))
        bests = tuple(jnp.where(u, v, bb)
                      for u, v, bb in zip(upds, phases, bests))
        bidxs = tuple(jnp.where(u, cf, bi) for u, bi in zip(upds, bidxs))
        return bests, bidxs

    best0 = tuple(jnp.full((SRC_HW, OUT_W), -jnp.inf, f32) for _ in range(4))
    bidx0 = tuple(jnp.zeros((SRC_HW, OUT_W), f32) for _ in range(4))
    _, bidxs = jax.lax.fori_loop(0, nchan, ch_body, (best0, bidx0))
    segf_blk = jnp.stack(bidxs, axis=1).reshape(OUT_H, OUT_W)
    segf = jnp.dot(segf_blk, colperm_ref[...], preferred_element_type=f32)

    yi = jax.lax.broadcasted_iota(jnp.int32, (BAND, MH), 0).astype(f32)
    ky = jax.lax.broadcasted_iota(jnp.int32, (BAND, MH), 1).astype(f32)
    jx = jax.lax.broadcasted_iota(jnp.int32, (MW, OUT_W), 0).astype(f32)
    xi = jax.lax.broadcasted_iota(jnp.int32, (MW, OUT_W), 1).astype(f32)

    yi = jax.lax.broadcasted_iota(jnp.int32, (OUT_H, MH), 0).astype(f32)
    ky = jax.lax.broadcasted_iota(jnp.int32, (OUT_H, MH), 1).astype(f32)
    jx = jax.lax.broadcasted_iota(jnp.int32, (MW, OUT_W), 0).astype(f32)
    xi = jax.lax.broadcasted_iota(jnp.int32, (MW, OUT_W), 1).astype(f32)

    def cand(d):
        ymin = boxes_sm[b, d, 0].astype(f32)
        ymaxc = jnp.minimum(boxes_sm[b, d, 2] + 1, OUT_H).astype(f32)
        xmin = boxes_sm[b, d, 1].astype(f32)
        xmaxc = jnp.minimum(boxes_sm[b, d, 3] + 1, OUT_W).astype(f32)
        bh = jnp.maximum(ymaxc - ymin, 1.0)
        bw = jnp.maximum(xmaxc - xmin, 1.0)
        fy = ((yi - ymin) + 0.5) * (MH / bh)
        sy = jnp.clip(jnp.floor(fy), 0.0, float(MH - 1))
        oy = ((ky == sy) & (yi >= ymin) & (yi < ymaxc)).astype(bf16)
        fx = ((xi - xmin) + 0.5) * (MW / bw)
        sx = jnp.clip(jnp.floor(fx), 0.0, float(MW - 1))
        oxt = ((jx == sx) & (xi >= xmin) & (xi < xmaxc)).astype(bf16)
        bm = (m_ref[0, d] > MASK_THR).astype(bf16)
        q = jnp.dot(oy, bm, preferred_element_type=f32)
        cov = jnp.dot(q.astype(bf16), oxt,
                      preferred_element_type=f32)
        e = encv_ref[pl.ds(d, 1), :]
        return jnp.where(cov > 0.5, e, VOID_ENC)

    quarter = (nreal + 3) // 4

    def det_body(d, encmin):
        c01 = jnp.minimum(cand(d), cand(d + quarter))
        c23 = jnp.minimum(cand(d + 2 * quarter), cand(d + 3 * quarter))
        return jnp.minimum(encmin, jnp.minimum(c01, c23))

    encmin = jax.lax.fori_loop(
        0, quarter, det_body, jnp.full((OUT_H, OUT_W), VOID_ENC, f32))

    found = encmin < VOID_ENC
    r = jnp.floor(encmin * (1.0 / 65536.0))
    rem = encmin - r * 65536.0
    cls = jnp.floor(rem * (1.0 / 128.0))
    dd = rem - cls * 128.0
    catf = jnp.where(found, cls, 0.0)
    instf = jnp.where(found, dd + 1.0, -1.0)
    stuff = (segf != 0.0) & (segf != 1.0)
    catf = jnp.where((~found) & stuff, segf + STUFF_OFFSET, catf)
    cat_ref[0] = catf.astype(jnp.int32)
    inst_ref[0] = instf.astype(jnp.int32)


def _run(detection_scores, detection_classes, detection_boxes,
         detection_masks, segmentation_outputs, interpret):
    B, N = detection_scores.shape
    C = segmentation_outputs.shape[-1]

    pad = NPAD - N
    scores = jnp.pad(detection_scores, ((0, 0), (0, pad)),
                     constant_values=-1.0)
    classes = jnp.pad(detection_classes, ((0, 0), (0, pad)))
    boxes = jnp.pad(detection_boxes, ((0, 0), (0, pad), (0, 0)))
    masks = jnp.pad(detection_masks, ((0, 0), (0, pad), (0, 0), (0, 0)))
    boxes_i = boxes.astype(jnp.int32)

    sa = scores.reshape(B, NPAD, 1)
    sb = scores.reshape(B, 1, NPAD)
    cls_a = classes.reshape(B, NPAD, 1)
    seg_t = jnp.transpose(segmentation_outputs, (0, 3, 1, 2))

    cp = np.zeros((OUT_W, OUT_W), np.float32)
    qq, jj = np.meshgrid(np.arange(4), np.arange(SRC_HW), indexing='ij')
    cp[SRC_HW * qq.ravel() + jj.ravel(), 4 * jj.ravel() + qq.ravel()] = 1.0
    colperm = jnp.asarray(cp)

    grid = (B,)
    kern = functools.partial(_panoptic_kernel, N, C)
    cat, inst = pl.pallas_call(
        kern,
        grid=grid,
        in_specs=[
            pl.BlockSpec((1, NPAD, 1), lambda b: (b, 0, 0)),
            pl.BlockSpec((1, 1, NPAD), lambda b: (b, 0, 0)),
            pl.BlockSpec((1, NPAD, 1), lambda b: (b, 0, 0)),
            pl.BlockSpec((1, NPAD, MH, MW), lambda b: (b, 0, 0, 0)),
            pl.BlockSpec((1, C, SRC_HW, SRC_HW), lambda b: (b, 0, 0, 0)),
            pl.BlockSpec((OUT_W, OUT_W), lambda b: (0, 0)),
            pl.BlockSpec(memory_space=pltpu.SMEM),
            pl.BlockSpec(memory_space=pltpu.SMEM),
        ],
        out_specs=[
            pl.BlockSpec((1, OUT_H, OUT_W), lambda b: (b, 0, 0)),
            pl.BlockSpec((1, OUT_H, OUT_W), lambda b: (b, 0, 0)),
        ],
        out_shape=[
            jax.ShapeDtypeStruct((B, OUT_H, OUT_W), jnp.int32),
            jax.ShapeDtypeStruct((B, OUT_H, OUT_W), jnp.int32),
        ],
        scratch_shapes=[pltpu.VMEM((NPAD, 1), jnp.float32),
                        pltpu.VMEM((OUT_H, OUT_W), jnp.float32)],
        interpret=interpret,
    )(sa, sb, cls_a, masks, seg_t, colperm, scores, boxes_i)
    return cat, inst


def kernel(detection_scores, detection_classes, detection_boxes,
           detection_masks, segmentation_outputs):
    return _run(detection_scores, detection_classes, detection_boxes,
                detection_masks, segmentation_outputs, False)


# speedup vs baseline: 1.8173x; 1.1266x over previous
import functools

import numpy as np
import jax
import jax.numpy as jnp
from jax.experimental import pallas as pl
from jax.experimental.pallas import tpu as pltpu

OUT_H = 512
OUT_W = 512
SRC_HW = 128
MH = 28
MW = 28
NPAD = 128
BAND = 128
STUFF_OFFSET = 90.0
MASK_THR = 0.5
SCORE_THR = 0.05
VOID_ENC = 6553600.0


def _interp_matrix(out_size: int, in_size: int) -> np.ndarray:
    i = np.arange(out_size, dtype=np.float32)
    src = (i + 0.5) * (in_size / out_size) - 0.5
    j = np.arange(in_size, dtype=np.float32)
    w = np.maximum(0.0, 1.0 - np.abs(j[None, :] - src[:, None])).astype(np.float32)
    w = w / np.sum(w, axis=1, keepdims=True)
    return w.astype(np.float32)


def _panoptic_kernel(nreal, nchan, sa_ref, sb_ref, cls_ref, m_ref, seg_ref,
                     colperm_ref, scores_sm, boxes_sm, cat_ref, inst_ref,
                     encv_ref, encmin_ref):
    f32 = jnp.float32
    bf16 = jnp.bfloat16
    b = pl.program_id(0)

    si = sa_ref[0]
    sj = sb_ref[0]
    ii = jax.lax.broadcasted_iota(jnp.int32, (NPAD, NPAD), 0)
    jj = jax.lax.broadcasted_iota(jnp.int32, (NPAD, NPAD), 1)
    beats = (sj > si) | ((sj == si) & (jj < ii))
    ranks = jnp.sum(beats.astype(f32), axis=1, keepdims=True)
    clsv = cls_ref[0]
    dv = jax.lax.broadcasted_iota(jnp.int32, (NPAD, 1), 0).astype(f32)
    validv = si > SCORE_THR
    encv_ref[...] = jnp.where(validv, ranks * 65536.0 + clsv * 128.0 + dv,
                              VOID_ENC)

    srcm1 = SRC_HW - 1

    def upsample(c):
        x = seg_ref[0, c]
        xm1 = jnp.concatenate([x[:, 0:1], x[:, 0:srcm1]], axis=1)
        xp1 = jnp.concatenate([x[:, 1:SRC_HW], x[:, srcm1:SRC_HW]], axis=1)
        t = jnp.concatenate(
            [0.375 * xm1 + 0.625 * x,
             0.125 * xm1 + 0.875 * x,
             0.875 * x + 0.125 * xp1,
             0.625 * x + 0.375 * xp1], axis=1)
        tm1 = jnp.concatenate([t[0:1], t[0:srcm1]], axis=0)
        tp1 = jnp.concatenate([t[1:SRC_HW], t[srcm1:SRC_HW]], axis=0)
        return (0.375 * tm1 + 0.625 * t,
                0.125 * tm1 + 0.875 * t,
                0.875 * t + 0.125 * tp1,
                0.625 * t + 0.375 * tp1)

    def ch_body(i, carry):
        bests, bidxs = carry
        c0 = 2 * i
        c1 = jnp.minimum(2 * i + 1, nchan - 1)
        ph0 = upsample(c0)
        ph1 = upsample(c1)
        cf0 = c0.astype(f32)
        cf1 = c1.astype(f32)
        pwin = tuple(v1 > v0 for v0, v1 in zip(ph0, ph1))
        pv = tuple(jnp.where(w, v1, v0)
                   for w, v0, v1 in zip(pwin, ph0, ph1))
        pc = tuple(jnp.where(w, cf1, cf0) for w in pwin)
        upds = tuple(v > bb for v, bb in zip(pv, bests))
        bests = tuple(jnp.where(u, v, bb)
                      for u, v, bb in zip(upds, pv, bests))
        bidxs = tuple(jnp.where(u, c, bi)
                      for u, c, bi in zip(upds, pc, bidxs))
        return bests, bidxs

    best0 = tuple(jnp.full((SRC_HW, OUT_W), -jnp.inf, f32) for _ in range(4))
    bidx0 = tuple(jnp.zeros((SRC_HW, OUT_W), f32) for _ in range(4))
    _, bidxs = jax.lax.fori_loop(0, (nchan + 1) // 2, ch_body,
                                 (best0, bidx0))
    segf_blk = jnp.stack(bidxs, axis=1).reshape(OUT_H, OUT_W)
    segf = jnp.dot(segf_blk, colperm_ref[...], preferred_element_type=f32)

    yi = jax.lax.broadcasted_iota(jnp.int32, (BAND, MH), 0).astype(f32)
    ky = jax.lax.broadcasted_iota(jnp.int32, (BAND, MH), 1).astype(f32)
    jx = jax.lax.broadcasted_iota(jnp.int32, (MW, OUT_W), 0).astype(f32)
    xi = jax.lax.broadcasted_iota(jnp.int32, (MW, OUT_W), 1).astype(f32)

    yi = jax.lax.broadcasted_iota(jnp.int32, (OUT_H, MH), 0).astype(f32)
    ky = jax.lax.broadcasted_iota(jnp.int32, (OUT_H, MH), 1).astype(f32)
    jx = jax.lax.broadcasted_iota(jnp.int32, (MW, OUT_W), 0).astype(f32)
    xi = jax.lax.broadcasted_iota(jnp.int32, (MW, OUT_W), 1).astype(f32)

    def cand(d):
        ymin = boxes_sm[b, d, 0].astype(f32)
        ymaxc = jnp.minimum(boxes_sm[b, d, 2] + 1, OUT_H).astype(f32)
        xmin = boxes_sm[b, d, 1].astype(f32)
        xmaxc = jnp.minimum(boxes_sm[b, d, 3] + 1, OUT_W).astype(f32)
        bh = jnp.maximum(ymaxc - ymin, 1.0)
        bw = jnp.maximum(xmaxc - xmin, 1.0)
        fy = ((yi - ymin) + 0.5) * (MH / bh)
        sy = jnp.clip(jnp.floor(fy), 0.0, float(MH - 1))
        oy = ((ky == sy) & (yi >= ymin) & (yi < ymaxc)).astype(bf16)
        fx = ((xi - xmin) + 0.5) * (MW / bw)
        sx = jnp.clip(jnp.floor(fx), 0.0, float(MW - 1))
        oxt = ((jx == sx) & (xi >= xmin) & (xi < xmaxc)).astype(bf16)
        bm = (m_ref[0, d] > MASK_THR).astype(bf16)
        q = jnp.dot(oy, bm, preferred_element_type=f32)
        cov = jnp.dot(q.astype(bf16), oxt,
                      preferred_element_type=f32)
        e = encv_ref[pl.ds(d, 1), :]
        return jnp.where(cov > 0.5, e, VOID_ENC)

    quarter = (nreal + 3) // 4

    def det_body(d, encmin):
        c01 = jnp.minimum(cand(d), cand(d + quarter))
        c23 = jnp.minimum(cand(d + 2 * quarter), cand(d + 3 * quarter))
        return jnp.minimum(encmin, jnp.minimum(c01, c23))

    encmin = jax.lax.fori_loop(
        0, quarter, det_body, jnp.full((OUT_H, OUT_W), VOID_ENC, f32))

    found = encmin < VOID_ENC
    r = jnp.floor(encmin * (1.0 / 65536.0))
    rem = encmin - r * 65536.0
    cls = jnp.floor(rem * (1.0 / 128.0))
    dd = rem - cls * 128.0
    catf = jnp.where(found, cls, 0.0)
    instf = jnp.where(found, dd + 1.0, -1.0)
    stuff = (segf != 0.0) & (segf != 1.0)
    catf = jnp.where((~found) & stuff, segf + STUFF_OFFSET, catf)
    cat_ref[0] = catf.astype(jnp.int32)
    inst_ref[0] = instf.astype(jnp.int32)


def _run(detection_scores, detection_classes, detection_boxes,
         detection_masks, segmentation_outputs, interpret):
    B, N = detection_scores.shape
    C = segmentation_outputs.shape[-1]

    pad = NPAD - N
    scores = jnp.pad(detection_scores, ((0, 0), (0, pad)),
                     constant_values=-1.0)
    classes = jnp.pad(detection_classes, ((0, 0), (0, pad)))
    boxes = jnp.pad(detection_boxes, ((0, 0), (0, pad), (0, 0)))
    masks = jnp.pad(detection_masks, ((0, 0), (0, pad), (0, 0), (0, 0)))
    boxes_i = boxes.astype(jnp.int32)

    sa = scores.reshape(B, NPAD, 1)
    sb = scores.reshape(B, 1, NPAD)
    cls_a = classes.reshape(B, NPAD, 1)
    seg_t = jnp.transpose(segmentation_outputs, (0, 3, 1, 2))

    cp = np.zeros((OUT_W, OUT_W), np.float32)
    qq, jj = np.meshgrid(np.arange(4), np.arange(SRC_HW), indexing='ij')
    cp[SRC_HW * qq.ravel() + jj.ravel(), 4 * jj.ravel() + qq.ravel()] = 1.0
    colperm = jnp.asarray(cp)

    grid = (B,)
    kern = functools.partial(_panoptic_kernel, N, C)
    cat, inst = pl.pallas_call(
        kern,
        grid=grid,
        in_specs=[
            pl.BlockSpec((1, NPAD, 1), lambda b: (b, 0, 0)),
            pl.BlockSpec((1, 1, NPAD), lambda b: (b, 0, 0)),
            pl.BlockSpec((1, NPAD, 1), lambda b: (b, 0, 0)),
            pl.BlockSpec((1, NPAD, MH, MW), lambda b: (b, 0, 0, 0)),
            pl.BlockSpec((1, C, SRC_HW, SRC_HW), lambda b: (b, 0, 0, 0)),
            pl.BlockSpec((OUT_W, OUT_W), lambda b: (0, 0)),
            pl.BlockSpec(memory_space=pltpu.SMEM),
            pl.BlockSpec(memory_space=pltpu.SMEM),
        ],
        out_specs=[
            pl.BlockSpec((1, OUT_H, OUT_W), lambda b: (b, 0, 0)),
            pl.BlockSpec((1, OUT_H, OUT_W), lambda b: (b, 0, 0)),
        ],
        out_shape=[
            jax.ShapeDtypeStruct((B, OUT_H, OUT_W), jnp.int32),
            jax.ShapeDtypeStruct((B, OUT_H, OUT_W), jnp.int32),
        ],
        scratch_shapes=[pltpu.VMEM((NPAD, 1), jnp.float32),
                        pltpu.VMEM((OUT_H, OUT_W), jnp.float32)],
        interpret=interpret,
    )(sa, sb, cls_a, masks, seg_t, colperm, scores, boxes_i)
    return cat, inst


def kernel(detection_scores, detection_classes, detection_boxes,
           detection_masks, segmentation_outputs):
    return _run(detection_scores, detection_classes, detection_boxes,
                detection_masks, segmentation_outputs, False)


# speedup vs baseline: 1.8928x; 1.0415x over previous
import functools

import numpy as np
import jax
import jax.numpy as jnp
from jax.experimental import pallas as pl
from jax.experimental.pallas import tpu as pltpu

OUT_H = 512
OUT_W = 512
SRC_HW = 128
MH = 28
MW = 28
NPAD = 128
BAND = 128
STUFF_OFFSET = 90.0
MASK_THR = 0.5
SCORE_THR = 0.05
VOID_ENC = 6553600.0


def _interp_matrix(out_size: int, in_size: int) -> np.ndarray:
    i = np.arange(out_size, dtype=np.float32)
    src = (i + 0.5) * (in_size / out_size) - 0.5
    j = np.arange(in_size, dtype=np.float32)
    w = np.maximum(0.0, 1.0 - np.abs(j[None, :] - src[:, None])).astype(np.float32)
    w = w / np.sum(w, axis=1, keepdims=True)
    return w.astype(np.float32)


def _panoptic_kernel(nreal, nchan, sa_ref, sb_ref, cls_ref, m_ref, seg_ref,
                     colperm_ref, scores_sm, boxes_sm, cat_ref, inst_ref,
                     encv_ref, encmin_ref):
    f32 = jnp.float32
    bf16 = jnp.bfloat16
    b = pl.program_id(0)

    si = sa_ref[0]
    sj = sb_ref[0]
    ii = jax.lax.broadcasted_iota(jnp.int32, (NPAD, NPAD), 0)
    jj = jax.lax.broadcasted_iota(jnp.int32, (NPAD, NPAD), 1)
    beats = (sj > si) | ((sj == si) & (jj < ii))
    ranks = jnp.sum(beats.astype(f32), axis=1, keepdims=True)
    clsv = cls_ref[0]
    dv = jax.lax.broadcasted_iota(jnp.int32, (NPAD, 1), 0).astype(f32)
    validv = si > SCORE_THR
    encv_ref[...] = jnp.where(validv, ranks * 65536.0 + clsv * 128.0 + dv,
                              VOID_ENC)

    srcm1 = SRC_HW - 1

    def upsample(c):
        x = seg_ref[0, c]
        xm1 = jnp.concatenate([x[:, 0:1], x[:, 0:srcm1]], axis=1)
        xp1 = jnp.concatenate([x[:, 1:SRC_HW], x[:, srcm1:SRC_HW]], axis=1)
        t = jnp.concatenate(
            [0.375 * xm1 + 0.625 * x,
             0.125 * xm1 + 0.875 * x,
             0.875 * x + 0.125 * xp1,
             0.625 * x + 0.375 * xp1], axis=1)
        tm1 = jnp.concatenate([t[0:1], t[0:srcm1]], axis=0)
        tp1 = jnp.concatenate([t[1:SRC_HW], t[srcm1:SRC_HW]], axis=0)
        return (0.375 * tm1 + 0.625 * t,
                0.125 * tm1 + 0.875 * t,
                0.875 * t + 0.125 * tp1,
                0.625 * t + 0.375 * tp1)

    def pairmax(v0, c0, v1, c1):
        w = tuple(b > a for a, b in zip(v0, v1))
        return (tuple(jnp.where(u, b, a) for u, a, b in zip(w, v0, v1)),
                tuple(jnp.where(u, b, a) for u, a, b in zip(w, c0, c1)))

    def ch_body(i, carry):
        bests, bidxs = carry
        cs = [jnp.minimum(4 * i + j, nchan - 1) for j in range(4)]
        phs = [upsample(c) for c in cs]
        cfs = [tuple(c.astype(f32) for _ in range(4)) for c in cs]
        v01, c01 = pairmax(phs[0], cfs[0], phs[1], cfs[1])
        v23, c23 = pairmax(phs[2], cfs[2], phs[3], cfs[3])
        v, cw = pairmax(v01, c01, v23, c23)
        bests, bidxs = pairmax(bests, bidxs, v, cw)
        return bests, bidxs

    best0 = tuple(jnp.full((SRC_HW, OUT_W), -jnp.inf, f32) for _ in range(4))
    bidx0 = tuple(jnp.zeros((SRC_HW, OUT_W), f32) for _ in range(4))
    _, bidxs = jax.lax.fori_loop(0, (nchan + 3) // 4, ch_body,
                                 (best0, bidx0))
    segf_blk = jnp.stack(bidxs, axis=1).reshape(OUT_H, OUT_W)
    segf = jnp.dot(segf_blk, colperm_ref[...], preferred_element_type=f32)

    yi = jax.lax.broadcasted_iota(jnp.int32, (BAND, MH), 0).astype(f32)
    ky = jax.lax.broadcasted_iota(jnp.int32, (BAND, MH), 1).astype(f32)
    jx = jax.lax.broadcasted_iota(jnp.int32, (MW, OUT_W), 0).astype(f32)
    xi = jax.lax.broadcasted_iota(jnp.int32, (MW, OUT_W), 1).astype(f32)

    yi = jax.lax.broadcasted_iota(jnp.int32, (OUT_H, MH), 0).astype(f32)
    ky = jax.lax.broadcasted_iota(jnp.int32, (OUT_H, MH), 1).astype(f32)
    jx = jax.lax.broadcasted_iota(jnp.int32, (MW, OUT_W), 0).astype(f32)
    xi = jax.lax.broadcasted_iota(jnp.int32, (MW, OUT_W), 1).astype(f32)

    def cand(d):
        ymin = boxes_sm[b, d, 0].astype(f32)
        ymaxc = jnp.minimum(boxes_sm[b, d, 2] + 1, OUT_H).astype(f32)
        xmin = boxes_sm[b, d, 1].astype(f32)
        xmaxc = jnp.minimum(boxes_sm[b, d, 3] + 1, OUT_W).astype(f32)
        bh = jnp.maximum(ymaxc - ymin, 1.0)
        bw = jnp.maximum(xmaxc - xmin, 1.0)
        fy = ((yi - ymin) + 0.5) * (MH / bh)
        sy = jnp.clip(jnp.floor(fy), 0.0, float(MH - 1))
        oy = ((ky == sy) & (yi >= ymin) & (yi < ymaxc)).astype(bf16)
        fx = ((xi - xmin) + 0.5) * (MW / bw)
        sx = jnp.clip(jnp.floor(fx), 0.0, float(MW - 1))
        oxt = ((jx == sx) & (xi >= xmin) & (xi < xmaxc)).astype(bf16)
        bm = (m_ref[0, d] > MASK_THR).astype(bf16)
        q = jnp.dot(oy, bm, preferred_element_type=f32)
        cov = jnp.dot(q.astype(bf16), oxt,
                      preferred_element_type=f32)
        e = encv_ref[pl.ds(d, 1), :]
        return jnp.where(cov > 0.5, e, VOID_ENC)

    quarter = (nreal + 3) // 4

    def det_body(d, encmin):
        c01 = jnp.minimum(cand(d), cand(d + quarter))
        c23 = jnp.minimum(cand(d + 2 * quarter), cand(d + 3 * quarter))
        return jnp.minimum(encmin, jnp.minimum(c01, c23))

    encmin = jax.lax.fori_loop(
        0, quarter, det_body, jnp.full((OUT_H, OUT_W), VOID_ENC, f32))

    found = encmin < VOID_ENC
    r = jnp.floor(encmin * (1.0 / 65536.0))
    rem = encmin - r * 65536.0
    cls = jnp.floor(rem * (1.0 / 128.0))
    dd = rem - cls * 128.0
    catf = jnp.where(found, cls, 0.0)
    instf = jnp.where(found, dd + 1.0, -1.0)
    stuff = (segf != 0.0) & (segf != 1.0)
    catf = jnp.where((~found) & stuff, segf + STUFF_OFFSET, catf)
    cat_ref[0] = catf.astype(jnp.int32)
    inst_ref[0] = instf.astype(jnp.int32)


def _run(detection_scores, detection_classes, detection_boxes,
         detection_masks, segmentation_outputs, interpret):
    B, N = detection_scores.shape
    C = segmentation_outputs.shape[-1]

    pad = NPAD - N
    scores = jnp.pad(detection_scores, ((0, 0), (0, pad)),
                     constant_values=-1.0)
    classes = jnp.pad(detection_classes, ((0, 0), (0, pad)))
    boxes = jnp.pad(detection_boxes, ((0, 0), (0, pad), (0, 0)))
    masks = jnp.pad(detection_masks, ((0, 0), (0, pad), (0, 0), (0, 0)))
    boxes_i = boxes.astype(jnp.int32)

    sa = scores.reshape(B, NPAD, 1)
    sb = scores.reshape(B, 1, NPAD)
    cls_a = classes.reshape(B, NPAD, 1)
    seg_t = jnp.transpose(segmentation_outputs, (0, 3, 1, 2))

    cp = np.zeros((OUT_W, OUT_W), np.float32)
    qq, jj = np.meshgrid(np.arange(4), np.arange(SRC_HW), indexing='ij')
    cp[SRC_HW * qq.ravel() + jj.ravel(), 4 * jj.ravel() + qq.ravel()] = 1.0
    colperm = jnp.asarray(cp)

    grid = (B,)
    kern = functools.partial(_panoptic_kernel, N, C)
    cat, inst = pl.pallas_call(
        kern,
        grid=grid,
        in_specs=[
            pl.BlockSpec((1, NPAD, 1), lambda b: (b, 0, 0)),
            pl.BlockSpec((1, 1, NPAD), lambda b: (b, 0, 0)),
            pl.BlockSpec((1, NPAD, 1), lambda b: (b, 0, 0)),
            pl.BlockSpec((1, NPAD, MH, MW), lambda b: (b, 0, 0, 0)),
            pl.BlockSpec((1, C, SRC_HW, SRC_HW), lambda b: (b, 0, 0, 0)),
            pl.BlockSpec((OUT_W, OUT_W), lambda b: (0, 0)),
            pl.BlockSpec(memory_space=pltpu.SMEM),
            pl.BlockSpec(memory_space=pltpu.SMEM),
        ],
        out_specs=[
            pl.BlockSpec((1, OUT_H, OUT_W), lambda b: (b, 0, 0)),
            pl.BlockSpec((1, OUT_H, OUT_W), lambda b: (b, 0, 0)),
        ],
        out_shape=[
            jax.ShapeDtypeStruct((B, OUT_H, OUT_W), jnp.int32),
            jax.ShapeDtypeStruct((B, OUT_H, OUT_W), jnp.int32),
        ],
        scratch_shapes=[pltpu.VMEM((NPAD, 1), jnp.float32),
                        pltpu.VMEM((OUT_H, OUT_W), jnp.float32)],
        interpret=interpret,
    )(sa, sb, cls_a, masks, seg_t, colperm, scores, boxes_i)
    return cat, inst


def kernel(detection_scores, detection_classes, detection_boxes,
           detection_masks, segmentation_outputs):
    return _run(detection_scores, detection_classes, detection_boxes,
                detection_masks, segmentation_outputs, False)


# speedup vs baseline: 1.8956x; 1.0015x over previous
import functools

import numpy as np
import jax
import jax.numpy as jnp
from jax.experimental import pallas as pl
from jax.experimental.pallas import tpu as pltpu

OUT_H = 512
OUT_W = 512
SRC_HW = 128
MH = 28
MW = 28
NPAD = 128
STUFF_OFFSET = 90.0
MASK_THR = 0.5
SCORE_THR = 0.05
VOID_ENC = 6553600.0


def _panoptic_kernel(nreal, nchan, sa_ref, sb_ref, cls_ref, m_ref, seg_ref,
                     colperm_ref, scores_sm, boxes_sm, cat_ref, inst_ref,
                     encv_ref):
    f32 = jnp.float32
    bf16 = jnp.bfloat16
    b = pl.program_id(0)

    si = sa_ref[0]
    sj = sb_ref[0]
    ii = jax.lax.broadcasted_iota(jnp.int32, (NPAD, NPAD), 0)
    jj = jax.lax.broadcasted_iota(jnp.int32, (NPAD, NPAD), 1)
    beats = (sj > si) | ((sj == si) & (jj < ii))
    ranks = jnp.sum(beats.astype(f32), axis=1, keepdims=True)
    clsv = cls_ref[0]
    dv = jax.lax.broadcasted_iota(jnp.int32, (NPAD, 1), 0).astype(f32)
    validv = si > SCORE_THR
    encv_ref[...] = jnp.where(validv, ranks * 65536.0 + clsv * 128.0 + dv,
                              VOID_ENC)

    srcm1 = SRC_HW - 1

    def upsample(c):
        x = seg_ref[0, c]
        xm1 = jnp.concatenate([x[:, 0:1], x[:, 0:srcm1]], axis=1)
        xp1 = jnp.concatenate([x[:, 1:SRC_HW], x[:, srcm1:SRC_HW]], axis=1)
        t = jnp.concatenate(
            [0.375 * xm1 + 0.625 * x,
             0.125 * xm1 + 0.875 * x,
             0.875 * x + 0.125 * xp1,
             0.625 * x + 0.375 * xp1], axis=1)
        tm1 = jnp.concatenate([t[0:1], t[0:srcm1]], axis=0)
        tp1 = jnp.concatenate([t[1:SRC_HW], t[srcm1:SRC_HW]], axis=0)
        return (0.375 * tm1 + 0.625 * t,
                0.125 * tm1 + 0.875 * t,
                0.875 * t + 0.125 * tp1,
                0.625 * t + 0.375 * tp1)

    def pairmax(v0, c0, v1, c1):
        w = tuple(b > a for a, b in zip(v0, v1))
        return (tuple(jnp.where(u, b, a) for u, a, b in zip(w, v0, v1)),
                tuple(jnp.where(u, b, a) for u, a, b in zip(w, c0, c1)))

    def ch_body(i, carry):
        bests, bidxs = carry
        cs = [jnp.minimum(4 * i + j, nchan - 1) for j in range(4)]
        phs = [upsample(c) for c in cs]
        cfs = [tuple(c.astype(f32) for _ in range(4)) for c in cs]
        v01, c01 = pairmax(phs[0], cfs[0], phs[1], cfs[1])
        v23, c23 = pairmax(phs[2], cfs[2], phs[3], cfs[3])
        v, cw = pairmax(v01, c01, v23, c23)
        bests, bidxs = pairmax(bests, bidxs, v, cw)
        return bests, bidxs

    best0 = tuple(jnp.full((SRC_HW, OUT_W), -jnp.inf, f32) for _ in range(4))
    bidx0 = tuple(jnp.zeros((SRC_HW, OUT_W), f32) for _ in range(4))
    _, bidxs = jax.lax.fori_loop(0, (nchan + 3) // 4, ch_body,
                                 (best0, bidx0))
    segf_blk = jnp.stack(bidxs, axis=1).reshape(OUT_H, OUT_W)
    segf = jnp.dot(segf_blk, colperm_ref[...], preferred_element_type=f32)

    yi = jax.lax.broadcasted_iota(jnp.int32, (OUT_H, MH), 0).astype(f32)
    ky = jax.lax.broadcasted_iota(jnp.int32, (OUT_H, MH), 1).astype(f32)
    jx = jax.lax.broadcasted_iota(jnp.int32, (MW, OUT_W), 0).astype(f32)
    xi = jax.lax.broadcasted_iota(jnp.int32, (MW, OUT_W), 1).astype(f32)

    def cand(d):
        ymin = boxes_sm[b, d, 0].astype(f32)
        ymaxc = jnp.minimum(boxes_sm[b, d, 2] + 1, OUT_H).astype(f32)
        xmin = boxes_sm[b, d, 1].astype(f32)
        xmaxc = jnp.minimum(boxes_sm[b, d, 3] + 1, OUT_W).astype(f32)
        bh = jnp.maximum(ymaxc - ymin, 1.0)
        bw = jnp.maximum(xmaxc - xmin, 1.0)
        fy = ((yi - ymin) + 0.5) * (MH / bh)
        sy = jnp.clip(jnp.floor(fy), 0.0, float(MH - 1))
        oy = ((ky == sy) & (yi >= ymin) & (yi < ymaxc)).astype(bf16)
        fx = ((xi - xmin) + 0.5) * (MW / bw)
        sx = jnp.clip(jnp.floor(fx), 0.0, float(MW - 1))
        oxt = ((jx == sx) & (xi >= xmin) & (xi < xmaxc)).astype(bf16)
        bm = (m_ref[0, d] > MASK_THR).astype(bf16)
        q = jnp.dot(oy, bm, preferred_element_type=f32)
        cov = jnp.dot(q.astype(bf16), oxt,
                      preferred_element_type=f32)
        e = encv_ref[pl.ds(d, 1), :]
        return jnp.where(cov > 0.5, e, VOID_ENC)

    quarter = (nreal + 3) // 4

    def det_body(d, encmin):
        c01 = jnp.minimum(cand(d), cand(d + quarter))
        c23 = jnp.minimum(cand(d + 2 * quarter), cand(d + 3 * quarter))
        return jnp.minimum(encmin, jnp.minimum(c01, c23))

    encmin = jax.lax.fori_loop(
        0, quarter, det_body, jnp.full((OUT_H, OUT_W), VOID_ENC, f32))

    found = encmin < VOID_ENC
    r = jnp.floor(encmin * (1.0 / 65536.0))
    rem = encmin - r * 65536.0
    cls = jnp.floor(rem * (1.0 / 128.0))
    dd = rem - cls * 128.0
    catf = jnp.where(found, cls, 0.0)
    instf = jnp.where(found, dd + 1.0, -1.0)
    stuff = (segf != 0.0) & (segf != 1.0)
    catf = jnp.where((~found) & stuff, segf + STUFF_OFFSET, catf)
    cat_ref[0] = catf.astype(jnp.int32)
    inst_ref[0] = instf.astype(jnp.int32)


def _run(detection_scores, detection_classes, detection_boxes,
         detection_masks, segmentation_outputs, interpret):
    B, N = detection_scores.shape
    C = segmentation_outputs.shape[-1]

    pad = NPAD - N
    scores = jnp.pad(detection_scores, ((0, 0), (0, pad)),
                     constant_values=-1.0)
    classes = jnp.pad(detection_classes, ((0, 0), (0, pad)))
    boxes = jnp.pad(detection_boxes, ((0, 0), (0, pad), (0, 0)))
    masks = jnp.pad(detection_masks, ((0, 0), (0, pad), (0, 0), (0, 0)))
    boxes_i = boxes.astype(jnp.int32)

    sa = scores.reshape(B, NPAD, 1)
    sb = scores.reshape(B, 1, NPAD)
    cls_a = classes.reshape(B, NPAD, 1)
    seg_t = jnp.transpose(segmentation_outputs, (0, 3, 1, 2))

    cp = np.zeros((OUT_W, OUT_W), np.float32)
    qq, jj = np.meshgrid(np.arange(4), np.arange(SRC_HW), indexing='ij')
    cp[SRC_HW * qq.ravel() + jj.ravel(), 4 * jj.ravel() + qq.ravel()] = 1.0
    colperm = jnp.asarray(cp)

    grid = (B,)
    kern = functools.partial(_panoptic_kernel, N, C)
    cat, inst = pl.pallas_call(
        kern,
        grid=grid,
        in_specs=[
            pl.BlockSpec((1, NPAD, 1), lambda b: (b, 0, 0)),
            pl.BlockSpec((1, 1, NPAD), lambda b: (b, 0, 0)),
            pl.BlockSpec((1, NPAD, 1), lambda b: (b, 0, 0)),
            pl.BlockSpec((1, NPAD, MH, MW), lambda b: (b, 0, 0, 0)),
            pl.BlockSpec((1, C, SRC_HW, SRC_HW), lambda b: (b, 0, 0, 0)),
            pl.BlockSpec((OUT_W, OUT_W), lambda b: (0, 0)),
            pl.BlockSpec(memory_space=pltpu.SMEM),
            pl.BlockSpec(memory_space=pltpu.SMEM),
        ],
        out_specs=[
            pl.BlockSpec((1, OUT_H, OUT_W), lambda b: (b, 0, 0)),
            pl.BlockSpec((1, OUT_H, OUT_W), lambda b: (b, 0, 0)),
        ],
        out_shape=[
            jax.ShapeDtypeStruct((B, OUT_H, OUT_W), jnp.int32),
            jax.ShapeDtypeStruct((B, OUT_H, OUT_W), jnp.int32),
        ],
        scratch_shapes=[pltpu.VMEM((NPAD, 1), jnp.float32)],
        interpret=interpret,
    )(sa, sb, cls_a, masks, seg_t, colperm, scores, boxes_i)
    return cat, inst


def kernel(detection_scores, detection_classes, detection_boxes,
           detection_masks, segmentation_outputs):
    return _run(detection_scores, detection_classes, detection_boxes,
                detection_masks, segmentation_outputs, False)
